# Initial kernel scaffold; baseline (speedup 1.0000x reference)
import jax
import jax.numpy as jnp
from jax.experimental import pallas as pl


def kernel(xyz):
    raise NotImplementedError("write your pallas kernel here")


# speedup vs baseline: 82.6871x; 82.6871x over previous
import jax
import jax.numpy as jnp
from jax.experimental import pallas as pl

_ROW_BLOCK = 256
_ALPHA = 5.0
_BIG = 3.0e38


def _dist_kernel(rows_ref, cols_ref, out_ref):
    rows = rows_ref[0]
    cols = cols_ref[0]
    rb = rows.astype(jnp.bfloat16).astype(jnp.float32)
    cb = cols.astype(jnp.bfloat16).astype(jnp.float32)
    dot = rb[:, 0:1] * cb[0:1, :]
    dot = dot + rb[:, 1:2] * cb[1:2, :]
    dot = dot + rb[:, 2:3] * cb[2:3, :]
    sqr = (rows[:, 0:1] * rows[:, 0:1] + rows[:, 1:2] * rows[:, 1:2]
           + rows[:, 2:3] * rows[:, 2:3])
    sqc = (cols[0:1, :] * cols[0:1, :] + cols[1:2, :] * cols[1:2, :]
           + cols[2:3, :] * cols[2:3, :])
    d2 = (sqr + sqc) - 2.0 * dot
    m1 = jnp.min(d2, axis=1, keepdims=True)
    eq = d2 == m1
    cnt = jnp.sum(jnp.where(eq, 1.0, 0.0), axis=1, keepdims=True)
    gt = jnp.min(jnp.where(eq, _BIG, d2), axis=1, keepdims=True)
    m2 = jnp.where(cnt >= 2.0, m1, gt)
    dist = jnp.sqrt(jnp.maximum(m1, 1e-12)) + jnp.sqrt(jnp.maximum(m2, 1e-12))
    out_ref[0] = dist


def _loss_kernel(dist_ref, out_ref):
    dist = dist_ref[...]
    avg = jnp.mean(dist, axis=1, keepdims=True)
    masked = jnp.where(dist > avg * _ALPHA, dist, 0.0)
    out_ref[...] = jnp.sum(masked).reshape(1, 1)


def kernel(xyz):
    B, N, _ = xyz.shape
    R = _ROW_BLOCK
    xt = jnp.transpose(xyz, (0, 2, 1))
    dist = pl.pallas_call(
        _dist_kernel,
        grid=(B, N // R),
        in_specs=[
            pl.BlockSpec((1, R, 3), lambda b, i: (b, i, 0)),
            pl.BlockSpec((1, 3, N), lambda b, i: (b, 0, 0)),
        ],
        out_specs=pl.BlockSpec((1, R, 1), lambda b, i: (b, i, 0)),
        out_shape=jax.ShapeDtypeStruct((B, N, 1), jnp.float32),
    )(xyz, xt)
    loss = pl.pallas_call(
        _loss_kernel,
        out_shape=jax.ShapeDtypeStruct((1, 1), jnp.float32),
    )(dist.reshape(B, N))
    return loss[0, 0]


# speedup vs baseline: 98.3019x; 1.1888x over previous
import jax
import jax.numpy as jnp
from jax.experimental import pallas as pl

_ROW_BLOCK = 256
_K_PAD = 128
_ALPHA = 5.0
_BIG = 3.0e38


def _dist_kernel(rows_ref, cols_ref, browe_ref, bcole_ref, out_ref):
    rows = rows_ref[0]
    cols = cols_ref[0]
    g = jax.lax.dot_general(
        browe_ref[0], bcole_ref[0],
        (((1,), (0,)), ((), ())),
        preferred_element_type=jnp.float32,
    )
    sqr = (rows[:, 0:1] * rows[:, 0:1] + rows[:, 1:2] * rows[:, 1:2]
           + rows[:, 2:3] * rows[:, 2:3])
    sqc = (cols[0:1, :] * cols[0:1, :] + cols[1:2, :] * cols[1:2, :]
           + cols[2:3, :] * cols[2:3, :])
    d2 = (sqr + sqc) - 2.0 * g
    m1 = jnp.min(d2, axis=1, keepdims=True)
    eq = d2 == m1
    cnt = jnp.sum(jnp.where(eq, 1.0, 0.0), axis=1, keepdims=True)
    gt = jnp.min(jnp.where(eq, _BIG, d2), axis=1, keepdims=True)
    m2 = jnp.where(cnt >= 2.0, m1, gt)
    dist = jnp.sqrt(jnp.maximum(m1, 1e-12)) + jnp.sqrt(jnp.maximum(m2, 1e-12))
    out_ref[0] = dist


def _loss_kernel(dist_ref, out_ref):
    dist = dist_ref[...]
    avg = jnp.mean(dist, axis=1, keepdims=True)
    masked = jnp.where(dist > avg * _ALPHA, dist, 0.0)
    out_ref[...] = jnp.sum(masked).reshape(1, 1)


def kernel(xyz):
    B, N, _ = xyz.shape
    R = _ROW_BLOCK
    xt = jnp.transpose(xyz, (0, 2, 1))
    xb = xyz.astype(jnp.bfloat16)
    brows = jnp.pad(xb, ((0, 0), (0, 0), (0, _K_PAD - 3)))
    bcols = jnp.transpose(brows, (0, 2, 1))
    dist = pl.pallas_call(
        _dist_kernel,
        grid=(B, N // R),
        in_specs=[
            pl.BlockSpec((1, R, 3), lambda b, i: (b, i, 0)),
            pl.BlockSpec((1, 3, N), lambda b, i: (b, 0, 0)),
            pl.BlockSpec((1, R, _K_PAD), lambda b, i: (b, i, 0)),
            pl.BlockSpec((1, _K_PAD, N), lambda b, i: (b, 0, 0)),
        ],
        out_specs=pl.BlockSpec((1, R, 1), lambda b, i: (b, i, 0)),
        out_shape=jax.ShapeDtypeStruct((B, N, 1), jnp.float32),
    )(xyz, xt, brows, bcols)
    loss = pl.pallas_call(
        _loss_kernel,
        out_shape=jax.ShapeDtypeStruct((1, 1), jnp.float32),
    )(dist.reshape(B, N))
    return loss[0, 0]


# speedup vs baseline: 105.9884x; 1.0782x over previous
import jax
import jax.numpy as jnp
from jax.experimental import pallas as pl

_ROW_BLOCK = 256
_K_PAD = 128
_ALPHA = 5.0
_BIG = 3.0e38


def _dist_kernel(rows_ref, sq_ref, brow_ref, ball_ref, out_ref):
    rows = rows_ref[0]
    g = jax.lax.dot_general(
        brow_ref[0], ball_ref[0],
        (((1,), (1,)), ((), ())),
        preferred_element_type=jnp.float32,
    )
    sqr = (rows[:, 0:1] * rows[:, 0:1] + rows[:, 1:2] * rows[:, 1:2]
           + rows[:, 2:3] * rows[:, 2:3])
    sqc = sq_ref[0]
    d2 = (sqr + sqc) - 2.0 * g
    m1 = jnp.min(d2, axis=1, keepdims=True)
    eq = d2 == m1
    cnt = jnp.sum(jnp.where(eq, 1.0, 0.0), axis=1, keepdims=True)
    gt = jnp.min(jnp.where(eq, _BIG, d2), axis=1, keepdims=True)
    m2 = jnp.where(cnt >= 2.0, m1, gt)
    dist = jnp.sqrt(jnp.maximum(m1, 1e-12)) + jnp.sqrt(jnp.maximum(m2, 1e-12))
    out_ref[0] = dist


def _loss_kernel(dist_ref, out_ref):
    dist = dist_ref[...]
    avg = jnp.mean(dist, axis=1, keepdims=True)
    masked = jnp.where(dist > avg * _ALPHA, dist, 0.0)
    out_ref[...] = jnp.sum(masked).reshape(1, 1)


def kernel(xyz):
    B, N, _ = xyz.shape
    R = _ROW_BLOCK
    sq = jnp.sum(xyz * xyz, axis=-1)[:, None, :]
    brows = jnp.pad(xyz.astype(jnp.bfloat16),
                    ((0, 0), (0, 0), (0, _K_PAD - 3)))
    dist = pl.pallas_call(
        _dist_kernel,
        grid=(B, N // R),
        in_specs=[
            pl.BlockSpec((1, R, 3), lambda b, i: (b, i, 0)),
            pl.BlockSpec((1, 1, N), lambda b, i: (b, 0, 0)),
            pl.BlockSpec((1, R, _K_PAD), lambda b, i: (b, i, 0)),
            pl.BlockSpec((1, N, _K_PAD), lambda b, i: (b, 0, 0)),
        ],
        out_specs=pl.BlockSpec((1, R, 1), lambda b, i: (b, i, 0)),
        out_shape=jax.ShapeDtypeStruct((B, N, 1), jnp.float32),
    )(xyz, sq, brows, brows)
    loss = pl.pallas_call(
        _loss_kernel,
        out_shape=jax.ShapeDtypeStruct((1, 1), jnp.float32),
    )(dist.reshape(B, N))
    return loss[0, 0]


# speedup vs baseline: 121.9941x; 1.1510x over previous
import jax
import jax.numpy as jnp
from jax.experimental import pallas as pl

_ROW_BLOCK = 256
_K_PAD = 128
_ALPHA = 5.0
_BIG = 3.0e38


def _dist_kernel(rows_ref, sq_ref, brow_ref, ball_ref, out_ref):
    rows = rows_ref[0]
    g = jax.lax.dot_general(
        brow_ref[0], ball_ref[0],
        (((1,), (1,)), ((), ())),
        preferred_element_type=jnp.float32,
    )
    sqr = (rows[:, 0:1] * rows[:, 0:1] + rows[:, 1:2] * rows[:, 1:2]
           + rows[:, 2:3] * rows[:, 2:3])
    sqc = sq_ref[0]
    d2 = (sqr + sqc) - 2.0 * g
    w = d2.shape[1] // 2
    m1v = jnp.minimum(d2[:, :w], d2[:, w:])
    m2v = jnp.maximum(d2[:, :w], d2[:, w:])
    w //= 2
    while w >= 128:
        a1, b1 = m1v[:, :w], m1v[:, w:]
        a2, b2 = m2v[:, :w], m2v[:, w:]
        nhi = jnp.maximum(a1, b1)
        m1v = jnp.minimum(a1, b1)
        m2v = jnp.minimum(jnp.minimum(a2, b2), nhi)
        w //= 2
    m1 = jnp.min(m1v, axis=1, keepdims=True)
    eqv = m1v == m1
    cntv = jnp.sum(jnp.where(eqv, 1.0, 0.0), axis=1, keepdims=True)
    gtv = jnp.min(jnp.where(eqv, _BIG, m1v), axis=1, keepdims=True)
    partner = jnp.min(jnp.where(eqv, m2v, _BIG), axis=1, keepdims=True)
    sec = jnp.where(cntv >= 2.0, m1, gtv)
    m2 = jnp.minimum(sec, partner)
    dist = jnp.sqrt(jnp.maximum(m1, 1e-12)) + jnp.sqrt(jnp.maximum(m2, 1e-12))
    out_ref[0] = dist


def _loss_kernel(dist_ref, out_ref):
    dist = dist_ref[...]
    avg = jnp.mean(dist, axis=1, keepdims=True)
    masked = jnp.where(dist > avg * _ALPHA, dist, 0.0)
    out_ref[...] = jnp.sum(masked).reshape(1, 1)


def kernel(xyz):
    B, N, _ = xyz.shape
    R = _ROW_BLOCK
    sq = jnp.sum(xyz * xyz, axis=-1)[:, None, :]
    brows = xyz.astype(jnp.bfloat16)
    dist = pl.pallas_call(
        _dist_kernel,
        grid=(B, N // R),
        in_specs=[
            pl.BlockSpec((1, R, 3), lambda b, i: (b, i, 0)),
            pl.BlockSpec((1, 1, N), lambda b, i: (b, 0, 0)),
            pl.BlockSpec((1, R, 3), lambda b, i: (b, i, 0)),
            pl.BlockSpec((1, N, 3), lambda b, i: (b, 0, 0)),
        ],
        out_specs=pl.BlockSpec((1, R, 1), lambda b, i: (b, i, 0)),
        out_shape=jax.ShapeDtypeStruct((B, N, 1), jnp.float32),
    )(xyz, sq, brows, brows)
    loss = pl.pallas_call(
        _loss_kernel,
        out_shape=jax.ShapeDtypeStruct((1, 1), jnp.float32),
    )(dist.reshape(B, N))
    return loss[0, 0]


# speedup vs baseline: 144.0326x; 1.1807x over previous
import functools
import jax
import jax.numpy as jnp
from jax.experimental import pallas as pl
from jax.experimental.pallas import tpu as pltpu

_ROW_BLOCK = 512
_ALPHA = 5.0
_BIG = 3.0e38


def _fused_kernel(rows_ref, sq_ref, all_ref, out_ref, dist_ref, *, nb):
    b = pl.program_id(0)
    i = pl.program_id(1)
    rows = rows_ref[0]
    brow = rows.astype(jnp.bfloat16)
    ball = all_ref[0].astype(jnp.bfloat16)
    g = jax.lax.dot_general(
        brow, ball,
        (((1,), (1,)), ((), ())),
        preferred_element_type=jnp.float32,
    )
    sqr = (rows[:, 0:1] * rows[:, 0:1] + rows[:, 1:2] * rows[:, 1:2]
           + rows[:, 2:3] * rows[:, 2:3])
    sqc = sq_ref[0]
    d2 = (sqr + sqc) - 2.0 * g
    w = d2.shape[1] // 2
    m1v = jnp.minimum(d2[:, :w], d2[:, w:])
    m2v = jnp.maximum(d2[:, :w], d2[:, w:])
    w //= 2
    while w >= 128:
        a1, b1 = m1v[:, :w], m1v[:, w:]
        a2, b2 = m2v[:, :w], m2v[:, w:]
        nhi = jnp.maximum(a1, b1)
        m1v = jnp.minimum(a1, b1)
        m2v = jnp.minimum(jnp.minimum(a2, b2), nhi)
        w //= 2
    m1 = jnp.min(m1v, axis=1, keepdims=True)
    eqv = m1v == m1
    cntv = jnp.sum(jnp.where(eqv, 1.0, 0.0), axis=1, keepdims=True)
    gtv = jnp.min(jnp.where(eqv, _BIG, m1v), axis=1, keepdims=True)
    partner = jnp.min(jnp.where(eqv, m2v, _BIG), axis=1, keepdims=True)
    sec = jnp.where(cntv >= 2.0, m1, gtv)
    m2 = jnp.minimum(sec, partner)
    dist = jnp.sqrt(jnp.maximum(m1, 1e-12)) + jnp.sqrt(jnp.maximum(m2, 1e-12))
    dist_ref[b, pl.ds(i * dist.shape[0], dist.shape[0])] = dist

    @pl.when(jnp.logical_and(b == dist_ref.shape[0] - 1, i == nb - 1))
    def _finalize():
        d = dist_ref[...]
        n = d.shape[1]
        avg = jnp.sum(d, axis=1, keepdims=True) / n
        masked = jnp.where(d > avg * _ALPHA, d, 0.0)
        out_ref[...] = jnp.sum(masked).reshape(1, 1)


def kernel(xyz):
    B, N, _ = xyz.shape
    R = _ROW_BLOCK
    nb = N // R
    sq = jnp.sum(xyz * xyz, axis=-1)[:, None, :]
    loss = pl.pallas_call(
        functools.partial(_fused_kernel, nb=nb),
        grid=(B, nb),
        in_specs=[
            pl.BlockSpec((1, R, 3), lambda b, i: (b, i, 0)),
            pl.BlockSpec((1, 1, N), lambda b, i: (b, 0, 0)),
            pl.BlockSpec((1, N, 3), lambda b, i: (b, 0, 0)),
        ],
        out_specs=pl.BlockSpec((1, 1), lambda b, i: (0, 0)),
        out_shape=jax.ShapeDtypeStruct((1, 1), jnp.float32),
        scratch_shapes=[pltpu.VMEM((B, N, 1), jnp.float32)],
    )(xyz, sq, xyz)
    return loss[0, 0]


# speedup vs baseline: 156.8029x; 1.0887x over previous
import functools
import jax
import jax.numpy as jnp
from jax.experimental import pallas as pl
from jax.experimental.pallas import tpu as pltpu

_ROW_BLOCK = 1024
_ALPHA = 5.0
_BIG = 3.0e38


def _fused_kernel(rows_ref, sq_ref, all_ref, out_ref, dist_ref, *, nb):
    b = pl.program_id(0)
    i = pl.program_id(1)
    rows = rows_ref[0]
    brow = rows.astype(jnp.bfloat16)
    brow2 = brow + brow
    ball = all_ref[0].astype(jnp.bfloat16)
    g2 = jax.lax.dot_general(
        brow2, ball,
        (((1,), (1,)), ((), ())),
        preferred_element_type=jnp.float32,
    )
    sqr = (rows[:, 0:1] * rows[:, 0:1] + rows[:, 1:2] * rows[:, 1:2]
           + rows[:, 2:3] * rows[:, 2:3])
    sqc = sq_ref[0]
    w = g2.shape[1] // 2
    d2a = (sqr + sqc[:, :w]) - g2[:, :w]
    d2b = (sqr + sqc[:, w:]) - g2[:, w:]
    m1v = jnp.minimum(d2a, d2b)
    m2v = jnp.maximum(d2a, d2b)
    w //= 2
    while w >= 128:
        a1, b1 = m1v[:, :w], m1v[:, w:]
        a2, b2 = m2v[:, :w], m2v[:, w:]
        nhi = jnp.maximum(a1, b1)
        m1v = jnp.minimum(a1, b1)
        m2v = jnp.minimum(jnp.minimum(a2, b2), nhi)
        w //= 2
    m1 = jnp.min(m1v, axis=1, keepdims=True)
    eqv = m1v == m1
    cntv = jnp.sum(jnp.where(eqv, 1.0, 0.0), axis=1, keepdims=True)
    gtv = jnp.min(jnp.where(eqv, _BIG, m1v), axis=1, keepdims=True)
    partner = jnp.min(jnp.where(eqv, m2v, _BIG), axis=1, keepdims=True)
    sec = jnp.where(cntv >= 2.0, m1, gtv)
    m2 = jnp.minimum(sec, partner)
    dist = jnp.sqrt(jnp.maximum(m1, 1e-12)) + jnp.sqrt(jnp.maximum(m2, 1e-12))
    dist_ref[b, pl.ds(i * dist.shape[0], dist.shape[0])] = dist

    @pl.when(jnp.logical_and(b == dist_ref.shape[0] - 1, i == nb - 1))
    def _finalize():
        d = dist_ref[...]
        n = d.shape[1]
        avg = jnp.sum(d, axis=1, keepdims=True) / n
        masked = jnp.where(d > avg * _ALPHA, d, 0.0)
        out_ref[...] = jnp.sum(masked).reshape(1, 1)


def kernel(xyz):
    B, N, _ = xyz.shape
    R = _ROW_BLOCK
    nb = N // R
    sq = jnp.sum(xyz * xyz, axis=-1)[:, None, :]
    loss = pl.pallas_call(
        functools.partial(_fused_kernel, nb=nb),
        grid=(B, nb),
        in_specs=[
            pl.BlockSpec((1, R, 3), lambda b, i: (b, i, 0)),
            pl.BlockSpec((1, 1, N), lambda b, i: (b, 0, 0)),
            pl.BlockSpec((1, N, 3), lambda b, i: (b, 0, 0)),
        ],
        out_specs=pl.BlockSpec((1, 1), lambda b, i: (0, 0)),
        out_shape=jax.ShapeDtypeStruct((1, 1), jnp.float32),
        scratch_shapes=[pltpu.VMEM((B, N, 1), jnp.float32)],
    )(xyz, sq, xyz)
    return loss[0, 0]
